# baseline (speedup 1.0000x reference)
import functools

import jax
import jax.numpy as jnp
from jax import lax
from jax.experimental import pallas as pl
from jax.experimental.pallas import tpu as pltpu
from jax.experimental.pallas import tpu_sc as plsc

_NC = 2
_NS = 16
_L = 16

_CH = 1024


def _tc_pre(ht, W_self0, W_neigh0, b0, W_self1, W_neigh1, b1, W_l, b_l, n):
    d = ht.shape[0]

    def body(h_ref, ws0, wn0, ws1, wn1, wl, b0r, b1r, blr, p0_ref, p1_ref, base_ref):
        wl_v = wl[...]
        v0 = jnp.dot(wn0[...], wl_v, preferred_element_type=jnp.float32)
        v1 = jnp.dot(wn1[...], wl_v, preferred_element_type=jnp.float32)
        u = jnp.dot(ws0[...] + ws1[...], wl_v, preferred_element_type=jnp.float32)
        c = jnp.sum((b0r[...] + b1r[...]) * wl_v[:, 0]) + blr[0]
        acc0 = jnp.zeros_like(p0_ref)
        acc1 = jnp.zeros_like(p0_ref)
        accb = jnp.zeros_like(p0_ref)
        for k in range(d):
            row = h_ref[k, :]
            acc0 = acc0 + row * v0[k, 0]
            acc1 = acc1 + row * v1[k, 0]
            accb = accb + row * u[k, 0]
        p0_ref[...] = acc0
        p1_ref[...] = acc1
        base_ref[...] = accb + c

    return pl.pallas_call(
        body,
        out_shape=[jax.ShapeDtypeStruct((n,), jnp.float32)] * 3,
    )(ht, W_self0, W_neigh0, W_self1, W_neigh1, W_l, b0, b1, b_l)


def _sc_edges(p0, p1, e0, e1, n, n_pad, n_edges):
    n_chunks = n_edges // _CH
    loop_n = pl.cdiv(n_chunks, _NS)
    n_groups = pl.cdiv(loop_n, 4)
    zlen = n_pad // _NS
    mesh = plsc.VectorSubcoreMesh(core_axis_name="c", subcore_axis_name="s")

    def body(p0_hbm, p1_hbm, e0_hbm, e1_hbm,
             s0_out, d0_out, s1_out, d1_out,
             table_v, src_v, idx_v, vals_v, s_sh,
             dma_s, dma_d, scat_s):
        cid = lax.axis_index("c")
        sid = lax.axis_index("s")

        def process(p_hbm, e_hbm, s_out, d_out):
            pltpu.sync_copy(p_hbm, table_v)

            def zb(i, _):
                vals_v[0][pl.ds(i * _L, _L)] = jnp.zeros((_L,), jnp.float32)
                for b in range(4):
                    vals_v[b][pl.ds(_CH + i * _L, _L)] = jnp.ones(
                        (_L,), jnp.float32)
                return _
            lax.fori_loop(0, _CH // _L, zb, None)

            for half in range(2):
                zoff = half * n_pad + sid * zlen
                o = 0
                while o < zlen:
                    l = min(_CH, zlen - o)
                    pltpu.sync_copy(vals_v[0].at[pl.ds(0, l)],
                                    s_sh.at[pl.ds(zoff + o, l)])
                    o += l
            plsc.subcore_barrier()

            m = (n_chunks - sid + _NS - 1) // _NS

            def src_slice(t):
                return e_hbm.at[pl.ds((sid + t * _NS) * _CH, _CH)]

            def dst_slice(t):
                return e_hbm.at[pl.ds(n_edges + (sid + t * _NS) * _CH, _CH)]

            def gather_chunk(sb, db):
                def g(i, _):
                    o0 = i * (4 * _L)
                    for u in range(4):
                        o = o0 + u * _L
                        idx = src_v[sb][pl.ds(o, _L)]
                        vals_v[db][pl.ds(o, _L)] = plsc.load_gather(
                            table_v, [idx])
                        dsts = idx_v[db][pl.ds(o, _L)]
                        idx_v[db][pl.ds(_CH + o, _L)] = dsts + n_pad
                    return _
                lax.fori_loop(0, _CH // (4 * _L), g, None)

            def dst_buf(b):
                return idx_v[b].at[pl.ds(0, _CH)]

            for t0 in range(2):
                @pl.when(t0 < m)
                def _(t0=t0):
                    pltpu.async_copy(src_slice(t0), src_v[t0 % 2], dma_s[t0 % 2])
                    pltpu.async_copy(dst_slice(t0), dst_buf(t0 % 4), dma_d[t0 % 4])

            def group(g, _):
                for b in range(4):
                    t = g * 4 + b
                    sb = b % 2
                    b2 = (b + 2) % 4

                    @pl.when(t < m)
                    def _(t=t, sb=sb, b=b):
                        pltpu.make_async_copy(src_slice(t), src_v[sb],
                                              dma_s[sb]).wait()
                        pltpu.make_async_copy(dst_slice(t), dst_buf(b),
                                              dma_d[b]).wait()
                        gather_chunk(sb, b)
                        pltpu.async_copy(vals_v[b], s_sh.at[idx_v[b]],
                                         scat_s[b], add=True)

                    @pl.when((t >= 2) & (t < m))
                    def _(b2=b2):
                        pltpu.make_async_copy(vals_v[b2], s_sh.at[idx_v[b2]],
                                              scat_s[b2]).wait()

                    @pl.when(t + 2 < m)
                    def _(t=t, sb=sb, b2=b2):
                        pltpu.async_copy(src_slice(t + 2), src_v[sb],
                                         dma_s[sb])
                        pltpu.async_copy(dst_slice(t + 2), dst_buf(b2),
                                         dma_d[b2])
                return _
            lax.fori_loop(0, n_groups, group, None)

            for b in range(4):
                for back in (1, 2):
                    @pl.when((m >= back) & ((m - back) % 4 == b))
                    def _(b=b):
                        pltpu.make_async_copy(vals_v[b], s_sh.at[idx_v[b]],
                                              scat_s[b]).wait()
            plsc.subcore_barrier()

            for half, out in ((0, s_out), (1, d_out)):
                zoff = half * n_pad + sid * zlen
                ooff = sid * zlen
                o = 0
                while o < zlen:
                    l = min(_CH, zlen - o)
                    pltpu.sync_copy(s_sh.at[pl.ds(zoff + o, l)],
                                    vals_v[0].at[pl.ds(0, l)])
                    pltpu.sync_copy(vals_v[0].at[pl.ds(0, l)],
                                    out.at[pl.ds(ooff + o, l)])
                    o += l

        @pl.when(cid == 0)
        def _():
            process(p0_hbm, e0_hbm, s0_out, d0_out)

        @pl.when(cid == 1)
        def _():
            process(p1_hbm, e1_hbm, s1_out, d1_out)

    f = pl.kernel(
        body,
        out_type=[jax.ShapeDtypeStruct((n_pad,), jnp.float32)] * 4,
        mesh=mesh,
        compiler_params=pltpu.CompilerParams(needs_layout_passes=False),
        scratch_types=[
            pltpu.VMEM((n,), jnp.float32),
            [pltpu.VMEM((_CH,), jnp.int32)] * 2,
            [pltpu.VMEM((2 * _CH,), jnp.int32)] * 4,
            [pltpu.VMEM((2 * _CH,), jnp.float32)] * 4,
            pltpu.VMEM_SHARED((2 * n_pad,), jnp.float32),
            [pltpu.SemaphoreType.DMA] * 2,
            [pltpu.SemaphoreType.DMA] * 4,
            [pltpu.SemaphoreType.DMA] * 4,
        ],
    )
    return f(p0, p1, e0, e1)


def _tc_post(base, s0, d0, s1, d1, n):
    def body(base_ref, s0_ref, d0_ref, s1_ref, d1_ref, out_ref):
        one = jnp.float32(1.0)
        m0 = s0_ref[pl.ds(0, n)] / jnp.maximum(d0_ref[pl.ds(0, n)], one)
        m1 = s1_ref[pl.ds(0, n)] / jnp.maximum(d1_ref[pl.ds(0, n)], one)
        out_ref[...] = base_ref[...] + m0 + m1

    return pl.pallas_call(
        body,
        out_shape=jax.ShapeDtypeStruct((n,), jnp.float32),
    )(base, s0, d0, s1, d1)


def kernel(h, edge_index_rel0, edge_index_rel1,
           W_self0, W_neigh0, b0,
           W_self1, W_neigh1, b1,
           W_l, b_l):
    n, d = h.shape
    n_edges = edge_index_rel0.shape[1]
    n_pad = ((n + 127) // 128) * 128

    p0, p1, base = _tc_pre(h.T, W_self0, W_neigh0, b0, W_self1, W_neigh1, b1,
                           W_l, b_l, n)
    s0, d0, s1, d1 = _sc_edges(
        p0, p1, edge_index_rel0.reshape(-1), edge_index_rel1.reshape(-1),
        n, n_pad, n_edges)
    out = _tc_post(base, s0, d0, s1, d1, n)
    return out.reshape(n, 1)


# speedup vs baseline: 1.2318x; 1.2318x over previous
import functools

import jax
import jax.numpy as jnp
from jax import lax
from jax.experimental import pallas as pl
from jax.experimental.pallas import tpu as pltpu
from jax.experimental.pallas import tpu_sc as plsc

_NC = 2
_NS = 16
_L = 16

_CH = 1280


def _tc_pre(ht, W_self0, W_neigh0, b0, W_self1, W_neigh1, b1, W_l, b_l, n):
    d = ht.shape[0]

    def body(h_ref, ws0, wn0, ws1, wn1, wl, b0r, b1r, blr, p0_ref, p1_ref, base_ref):
        wl_v = wl[...]
        v0 = jnp.dot(wn0[...], wl_v, preferred_element_type=jnp.float32)
        v1 = jnp.dot(wn1[...], wl_v, preferred_element_type=jnp.float32)
        u = jnp.dot(ws0[...] + ws1[...], wl_v, preferred_element_type=jnp.float32)
        c = jnp.sum((b0r[...] + b1r[...]) * wl_v[:, 0]) + blr[0]
        acc0 = jnp.zeros_like(p0_ref)
        acc1 = jnp.zeros_like(p0_ref)
        accb = jnp.zeros_like(p0_ref)
        for k in range(d):
            row = h_ref[k, :]
            acc0 = acc0 + row * v0[k, 0]
            acc1 = acc1 + row * v1[k, 0]
            accb = accb + row * u[k, 0]
        p0_ref[...] = acc0
        p1_ref[...] = acc1
        base_ref[...] = accb + c

    return pl.pallas_call(
        body,
        out_shape=[jax.ShapeDtypeStruct((n,), jnp.float32)] * 3,
    )(ht, W_self0, W_neigh0, W_self1, W_neigh1, W_l, b0, b1, b_l)


def _sc_edges(p0, p1, e0, e1, n, n_pad, n_edges):
    n_chunks = n_edges // _CH
    loop_n = pl.cdiv(n_chunks, _NS)
    n_groups = pl.cdiv(loop_n, 4)
    zlen = n_pad // _NS
    mesh = plsc.VectorSubcoreMesh(core_axis_name="c", subcore_axis_name="s")

    def body(p0_hbm, p1_hbm, e0_hbm, e1_hbm,
             s0_out, d0_out, s1_out, d1_out,
             table_v, src_v, dst_v, vals_v, ones_v, s_sh, d_sh,
             dma_s, dma_d, scat_s, scat_d):
        cid = lax.axis_index("c")
        sid = lax.axis_index("s")

        def process(p_hbm, e_hbm, s_out, d_out):
            pltpu.sync_copy(p_hbm, table_v)

            def zb(i, _):
                vals_v[0][pl.ds(i * _L, _L)] = jnp.zeros((_L,), jnp.float32)
                ones_v[pl.ds(i * _L, _L)] = jnp.ones((_L,), jnp.float32)
                return _
            lax.fori_loop(0, _CH // _L, zb, None)

            zoff = sid * zlen
            for acc in (s_sh, d_sh):
                o = 0
                while o < zlen:
                    l = min(_CH, zlen - o)
                    pltpu.sync_copy(vals_v[0].at[pl.ds(0, l)],
                                    acc.at[pl.ds(zoff + o, l)])
                    o += l
            plsc.subcore_barrier()

            m = (n_chunks - sid + _NS - 1) // _NS

            def src_slice(t):
                return e_hbm.at[pl.ds((sid + t * _NS) * _CH, _CH)]

            def dst_slice(t):
                return e_hbm.at[pl.ds(n_edges + (sid + t * _NS) * _CH, _CH)]

            def gather_chunk(sb, db):
                def g(i, _):
                    o0 = i * (4 * _L)
                    for u in range(4):
                        o = o0 + u * _L
                        idx = src_v[sb][pl.ds(o, _L)]
                        vals_v[db][pl.ds(o, _L)] = plsc.load_gather(
                            table_v, [idx])
                    return _
                lax.fori_loop(0, _CH // (4 * _L), g, None)

            for t0 in range(2):
                @pl.when(t0 < m)
                def _(t0=t0):
                    pltpu.async_copy(src_slice(t0), src_v[t0 % 2], dma_s[t0 % 2])
                    pltpu.async_copy(dst_slice(t0), dst_v[t0 % 4], dma_d[t0 % 4])

            def group(g, _):
                for b in range(4):
                    t = g * 4 + b
                    sb = b % 2
                    b2 = (b + 2) % 4

                    @pl.when(t < m)
                    def _(t=t, sb=sb, b=b):
                        pltpu.make_async_copy(src_slice(t), src_v[sb],
                                              dma_s[sb]).wait()
                        pltpu.make_async_copy(dst_slice(t), dst_v[b],
                                              dma_d[b]).wait()
                        gather_chunk(sb, b)
                        pltpu.async_copy(vals_v[b], s_sh.at[dst_v[b]],
                                         scat_s[b], add=True)
                        pltpu.async_copy(ones_v, d_sh.at[dst_v[b]],
                                         scat_d[b], add=True)

                    @pl.when((t >= 2) & (t < m))
                    def _(b2=b2):
                        pltpu.make_async_copy(vals_v[b2], s_sh.at[dst_v[b2]],
                                              scat_s[b2]).wait()
                        pltpu.make_async_copy(ones_v, d_sh.at[dst_v[b2]],
                                              scat_d[b2]).wait()

                    @pl.when(t + 2 < m)
                    def _(t=t, sb=sb, b2=b2):
                        pltpu.async_copy(src_slice(t + 2), src_v[sb],
                                         dma_s[sb])
                        pltpu.async_copy(dst_slice(t + 2), dst_v[b2],
                                         dma_d[b2])
                return _
            lax.fori_loop(0, n_groups, group, None)

            for b in range(4):
                for back in (1, 2):
                    @pl.when((m >= back) & ((m - back) % 4 == b))
                    def _(b=b):
                        pltpu.make_async_copy(vals_v[b], s_sh.at[dst_v[b]],
                                              scat_s[b]).wait()
                        pltpu.make_async_copy(ones_v, d_sh.at[dst_v[b]],
                                              scat_d[b]).wait()
            plsc.subcore_barrier()

            for acc, out in ((s_sh, s_out), (d_sh, d_out)):
                o = 0
                while o < zlen:
                    l = min(_CH, zlen - o)
                    pltpu.sync_copy(acc.at[pl.ds(zoff + o, l)],
                                    vals_v[0].at[pl.ds(0, l)])
                    pltpu.sync_copy(vals_v[0].at[pl.ds(0, l)],
                                    out.at[pl.ds(zoff + o, l)])
                    o += l

        @pl.when(cid == 0)
        def _():
            process(p0_hbm, e0_hbm, s0_out, d0_out)

        @pl.when(cid == 1)
        def _():
            process(p1_hbm, e1_hbm, s1_out, d1_out)

    f = pl.kernel(
        body,
        out_type=[jax.ShapeDtypeStruct((n_pad,), jnp.float32)] * 4,
        mesh=mesh,
        compiler_params=pltpu.CompilerParams(needs_layout_passes=False),
        scratch_types=[
            pltpu.VMEM((n,), jnp.float32),
            [pltpu.VMEM((_CH,), jnp.int32)] * 2,
            [pltpu.VMEM((_CH,), jnp.int32)] * 4,
            [pltpu.VMEM((_CH,), jnp.float32)] * 4,
            pltpu.VMEM((_CH,), jnp.float32),
            pltpu.VMEM_SHARED((n_pad,), jnp.float32),
            pltpu.VMEM_SHARED((n_pad,), jnp.float32),
            [pltpu.SemaphoreType.DMA] * 2,
            [pltpu.SemaphoreType.DMA] * 4,
            [pltpu.SemaphoreType.DMA] * 4,
            [pltpu.SemaphoreType.DMA] * 4,
        ],
    )
    return f(p0, p1, e0, e1)


def _tc_post(base, s0, d0, s1, d1, n):
    def body(base_ref, s0_ref, d0_ref, s1_ref, d1_ref, out_ref):
        one = jnp.float32(1.0)
        m0 = s0_ref[pl.ds(0, n)] / jnp.maximum(d0_ref[pl.ds(0, n)], one)
        m1 = s1_ref[pl.ds(0, n)] / jnp.maximum(d1_ref[pl.ds(0, n)], one)
        out_ref[...] = base_ref[...] + m0 + m1

    return pl.pallas_call(
        body,
        out_shape=jax.ShapeDtypeStruct((n,), jnp.float32),
    )(base, s0, d0, s1, d1)


def kernel(h, edge_index_rel0, edge_index_rel1,
           W_self0, W_neigh0, b0,
           W_self1, W_neigh1, b1,
           W_l, b_l):
    n, d = h.shape
    n_edges = edge_index_rel0.shape[1]
    n_pad = ((n + 127) // 128) * 128

    p0, p1, base = _tc_pre(h.T, W_self0, W_neigh0, b0, W_self1, W_neigh1, b1,
                           W_l, b_l, n)
    s0, d0, s1, d1 = _sc_edges(
        p0, p1, edge_index_rel0.reshape(-1), edge_index_rel1.reshape(-1),
        n, n_pad, n_edges)
    out = _tc_post(base, s0, d0, s1, d1, n)
    return out.reshape(n, 1)


# speedup vs baseline: 1.2364x; 1.0038x over previous
import functools

import jax
import jax.numpy as jnp
from jax import lax
from jax.experimental import pallas as pl
from jax.experimental.pallas import tpu as pltpu
from jax.experimental.pallas import tpu_sc as plsc

_NC = 2
_NS = 16
_L = 16

_CH = 1600


def _tc_pre(ht, W_self0, W_neigh0, b0, W_self1, W_neigh1, b1, W_l, b_l, n):
    d = ht.shape[0]

    def body(h_ref, ws0, wn0, ws1, wn1, wl, b0r, b1r, blr, p0_ref, p1_ref, base_ref):
        wl_v = wl[...]
        v0 = jnp.dot(wn0[...], wl_v, preferred_element_type=jnp.float32)
        v1 = jnp.dot(wn1[...], wl_v, preferred_element_type=jnp.float32)
        u = jnp.dot(ws0[...] + ws1[...], wl_v, preferred_element_type=jnp.float32)
        c = jnp.sum((b0r[...] + b1r[...]) * wl_v[:, 0]) + blr[0]
        acc0 = jnp.zeros_like(p0_ref)
        acc1 = jnp.zeros_like(p0_ref)
        accb = jnp.zeros_like(p0_ref)
        for k in range(d):
            row = h_ref[k, :]
            acc0 = acc0 + row * v0[k, 0]
            acc1 = acc1 + row * v1[k, 0]
            accb = accb + row * u[k, 0]
        p0_ref[...] = acc0
        p1_ref[...] = acc1
        base_ref[...] = accb + c

    return pl.pallas_call(
        body,
        out_shape=[jax.ShapeDtypeStruct((n,), jnp.float32)] * 3,
    )(ht, W_self0, W_neigh0, W_self1, W_neigh1, W_l, b0, b1, b_l)


def _sc_edges(p0, p1, e0, e1, n, n_pad, n_edges):
    n_chunks = n_edges // _CH
    loop_n = pl.cdiv(n_chunks, _NS)
    n_groups = pl.cdiv(loop_n, 4)
    zlen = n_pad // _NS
    mesh = plsc.VectorSubcoreMesh(core_axis_name="c", subcore_axis_name="s")

    def body(p0_hbm, p1_hbm, e0_hbm, e1_hbm,
             s0_out, d0_out, s1_out, d1_out,
             table_v, src_v, dst_v, vals_v, ones_v, s_sh, d_sh,
             dma_s, dma_d, scat_s, scat_d):
        cid = lax.axis_index("c")
        sid = lax.axis_index("s")

        def process(p_hbm, e_hbm, s_out, d_out):
            pltpu.sync_copy(p_hbm, table_v)

            def zb(i, _):
                vals_v[0][pl.ds(i * _L, _L)] = jnp.zeros((_L,), jnp.float32)
                ones_v[pl.ds(i * _L, _L)] = jnp.ones((_L,), jnp.float32)
                return _
            lax.fori_loop(0, _CH // _L, zb, None)

            zoff = sid * zlen
            for acc in (s_sh, d_sh):
                o = 0
                while o < zlen:
                    l = min(_CH, zlen - o)
                    pltpu.sync_copy(vals_v[0].at[pl.ds(0, l)],
                                    acc.at[pl.ds(zoff + o, l)])
                    o += l
            plsc.subcore_barrier()

            m = (n_chunks - sid + _NS - 1) // _NS

            def src_slice(t):
                return e_hbm.at[pl.ds((sid + t * _NS) * _CH, _CH)]

            def dst_slice(t):
                return e_hbm.at[pl.ds(n_edges + (sid + t * _NS) * _CH, _CH)]

            def gather_chunk(sb, db):
                def g(i, _):
                    o0 = i * (4 * _L)
                    for u in range(4):
                        o = o0 + u * _L
                        idx = src_v[sb][pl.ds(o, _L)]
                        vals_v[db][pl.ds(o, _L)] = plsc.load_gather(
                            table_v, [idx])
                    return _
                lax.fori_loop(0, _CH // (4 * _L), g, None)

            for t0 in range(2):
                @pl.when(t0 < m)
                def _(t0=t0):
                    pltpu.async_copy(src_slice(t0), src_v[t0 % 2], dma_s[t0 % 2])
                    pltpu.async_copy(dst_slice(t0), dst_v[t0 % 4], dma_d[t0 % 4])

            def group(g, _):
                for b in range(4):
                    t = g * 4 + b
                    sb = b % 2
                    b2 = (b + 2) % 4

                    @pl.when(t < m)
                    def _(t=t, sb=sb, b=b):
                        pltpu.make_async_copy(src_slice(t), src_v[sb],
                                              dma_s[sb]).wait()
                        pltpu.make_async_copy(dst_slice(t), dst_v[b],
                                              dma_d[b]).wait()
                        gather_chunk(sb, b)
                        pltpu.async_copy(vals_v[b], s_sh.at[dst_v[b]],
                                         scat_s[b], add=True)
                        pltpu.async_copy(ones_v, d_sh.at[dst_v[b]],
                                         scat_d[b], add=True)

                    @pl.when((t >= 2) & (t < m))
                    def _(b2=b2):
                        pltpu.make_async_copy(vals_v[b2], s_sh.at[dst_v[b2]],
                                              scat_s[b2]).wait()
                        pltpu.make_async_copy(ones_v, d_sh.at[dst_v[b2]],
                                              scat_d[b2]).wait()

                    @pl.when(t + 2 < m)
                    def _(t=t, sb=sb, b2=b2):
                        pltpu.async_copy(src_slice(t + 2), src_v[sb],
                                         dma_s[sb])
                        pltpu.async_copy(dst_slice(t + 2), dst_v[b2],
                                         dma_d[b2])
                return _
            lax.fori_loop(0, n_groups, group, None)

            for b in range(4):
                for back in (1, 2):
                    @pl.when((m >= back) & ((m - back) % 4 == b))
                    def _(b=b):
                        pltpu.make_async_copy(vals_v[b], s_sh.at[dst_v[b]],
                                              scat_s[b]).wait()
                        pltpu.make_async_copy(ones_v, d_sh.at[dst_v[b]],
                                              scat_d[b]).wait()
            plsc.subcore_barrier()

            for acc, out in ((s_sh, s_out), (d_sh, d_out)):
                o = 0
                while o < zlen:
                    l = min(_CH, zlen - o)
                    pltpu.sync_copy(acc.at[pl.ds(zoff + o, l)],
                                    vals_v[0].at[pl.ds(0, l)])
                    pltpu.sync_copy(vals_v[0].at[pl.ds(0, l)],
                                    out.at[pl.ds(zoff + o, l)])
                    o += l

        @pl.when(cid == 0)
        def _():
            process(p0_hbm, e0_hbm, s0_out, d0_out)

        @pl.when(cid == 1)
        def _():
            process(p1_hbm, e1_hbm, s1_out, d1_out)

    f = pl.kernel(
        body,
        out_type=[jax.ShapeDtypeStruct((n_pad,), jnp.float32)] * 4,
        mesh=mesh,
        compiler_params=pltpu.CompilerParams(needs_layout_passes=False),
        scratch_types=[
            pltpu.VMEM((n,), jnp.float32),
            [pltpu.VMEM((_CH,), jnp.int32)] * 2,
            [pltpu.VMEM((_CH,), jnp.int32)] * 4,
            [pltpu.VMEM((_CH,), jnp.float32)] * 4,
            pltpu.VMEM((_CH,), jnp.float32),
            pltpu.VMEM_SHARED((n_pad,), jnp.float32),
            pltpu.VMEM_SHARED((n_pad,), jnp.float32),
            [pltpu.SemaphoreType.DMA] * 2,
            [pltpu.SemaphoreType.DMA] * 4,
            [pltpu.SemaphoreType.DMA] * 4,
            [pltpu.SemaphoreType.DMA] * 4,
        ],
    )
    return f(p0, p1, e0, e1)


def _tc_post(base, s0, d0, s1, d1, n):
    def body(base_ref, s0_ref, d0_ref, s1_ref, d1_ref, out_ref):
        one = jnp.float32(1.0)
        m0 = s0_ref[pl.ds(0, n)] / jnp.maximum(d0_ref[pl.ds(0, n)], one)
        m1 = s1_ref[pl.ds(0, n)] / jnp.maximum(d1_ref[pl.ds(0, n)], one)
        out_ref[...] = base_ref[...] + m0 + m1

    return pl.pallas_call(
        body,
        out_shape=jax.ShapeDtypeStruct((n,), jnp.float32),
    )(base, s0, d0, s1, d1)


def kernel(h, edge_index_rel0, edge_index_rel1,
           W_self0, W_neigh0, b0,
           W_self1, W_neigh1, b1,
           W_l, b_l):
    n, d = h.shape
    n_edges = edge_index_rel0.shape[1]
    n_pad = ((n + 127) // 128) * 128

    p0, p1, base = _tc_pre(h.T, W_self0, W_neigh0, b0, W_self1, W_neigh1, b1,
                           W_l, b_l, n)
    s0, d0, s1, d1 = _sc_edges(
        p0, p1, edge_index_rel0.reshape(-1), edge_index_rel1.reshape(-1),
        n, n_pad, n_edges)
    out = _tc_post(base, s0, d0, s1, d1, n)
    return out.reshape(n, 1)


# speedup vs baseline: 1.2460x; 1.0077x over previous
import functools

import jax
import jax.numpy as jnp
from jax import lax
from jax.experimental import pallas as pl
from jax.experimental.pallas import tpu as pltpu
from jax.experimental.pallas import tpu_sc as plsc

_NC = 2
_NS = 16
_L = 16

_CH = 1600


def _tc_pre(ht, W_self0, W_neigh0, b0, W_self1, W_neigh1, b1, W_l, b_l, n):
    d = ht.shape[0]

    def body(h_ref, ws0, wn0, ws1, wn1, wl, b0r, b1r, blr, p0_ref, p1_ref, base_ref):
        wl_v = wl[...]
        v0 = jnp.dot(wn0[...], wl_v, preferred_element_type=jnp.float32)
        v1 = jnp.dot(wn1[...], wl_v, preferred_element_type=jnp.float32)
        u = jnp.dot(ws0[...] + ws1[...], wl_v, preferred_element_type=jnp.float32)
        c = jnp.sum((b0r[...] + b1r[...]) * wl_v[:, 0]) + blr[0]
        acc0 = jnp.zeros_like(p0_ref)
        acc1 = jnp.zeros_like(p0_ref)
        accb = jnp.zeros_like(p0_ref)
        for k in range(d):
            row = h_ref[k, :]
            acc0 = acc0 + row * v0[k, 0]
            acc1 = acc1 + row * v1[k, 0]
            accb = accb + row * u[k, 0]
        p0_ref[...] = acc0
        p1_ref[...] = acc1
        base_ref[...] = accb + c

    return pl.pallas_call(
        body,
        out_shape=[jax.ShapeDtypeStruct((n,), jnp.float32)] * 3,
    )(ht, W_self0, W_neigh0, W_self1, W_neigh1, W_l, b0, b1, b_l)


def _sc_edges(p0, p1, e0, e1, n, n_pad, n_edges):
    n_chunks = n_edges // _CH
    loop_n = pl.cdiv(n_chunks, _NS)
    n_groups = pl.cdiv(loop_n, 4)
    zlen = n_pad // _NS
    mesh = plsc.VectorSubcoreMesh(core_axis_name="c", subcore_axis_name="s")

    def body(p0_hbm, p1_hbm, e0_hbm, e1_hbm,
             s0_out, d0_out, s1_out, d1_out,
             table_v, src_v, dst_v, vals_v, ones_v, s_sh, d_sh,
             dma_s, dma_d, scat_s, scat_d):
        cid = lax.axis_index("c")
        sid = lax.axis_index("s")

        def process(p_hbm, e_hbm, s_out, d_out):
            pltpu.sync_copy(p_hbm, table_v)

            def zb(i, _):
                vals_v[0][pl.ds(i * _L, _L)] = jnp.zeros((_L,), jnp.float32)
                ones_v[pl.ds(i * _L, _L)] = jnp.ones((_L,), jnp.float32)
                return _
            lax.fori_loop(0, _CH // _L, zb, None)

            zoff = sid * zlen
            for acc in (s_sh, d_sh):
                o = 0
                while o < zlen:
                    l = min(_CH, zlen - o)
                    pltpu.sync_copy(vals_v[0].at[pl.ds(0, l)],
                                    acc.at[pl.ds(zoff + o, l)])
                    o += l
            plsc.subcore_barrier()

            m = (n_chunks - sid + _NS - 1) // _NS

            def src_slice(t):
                return e_hbm.at[pl.ds((sid + t * _NS) * _CH, _CH)]

            def dst_slice(t):
                return e_hbm.at[pl.ds(n_edges + (sid + t * _NS) * _CH, _CH)]

            def gather_chunk(sb, db):
                def g(i, _):
                    o0 = i * (4 * _L)
                    for u in range(4):
                        o = o0 + u * _L
                        idx = src_v[sb][pl.ds(o, _L)]
                        vals_v[db][pl.ds(o, _L)] = plsc.load_gather(
                            table_v, [idx])
                    return _
                lax.fori_loop(0, _CH // (4 * _L), g, None)

            for t0 in range(2):
                @pl.when(t0 < m)
                def _(t0=t0):
                    pltpu.async_copy(src_slice(t0), src_v[t0 % 2], dma_s[t0 % 2])
                    pltpu.async_copy(dst_slice(t0), dst_v[t0 % 4], dma_d[t0 % 4])

            def group(g, _):
                for b in range(4):
                    t = g * 4 + b
                    sb = b % 2
                    b2 = (b + 2) % 4

                    @pl.when(t < m)
                    def _(t=t, sb=sb, b=b):
                        pltpu.make_async_copy(src_slice(t), src_v[sb],
                                              dma_s[sb]).wait()
                        pltpu.make_async_copy(dst_slice(t), dst_v[b],
                                              dma_d[b]).wait()
                        pltpu.async_copy(ones_v, d_sh.at[dst_v[b]],
                                         scat_d[b], add=True)
                        gather_chunk(sb, b)
                        pltpu.async_copy(vals_v[b], s_sh.at[dst_v[b]],
                                         scat_s[b], add=True)

                    @pl.when((t >= 2) & (t < m))
                    def _(b2=b2):
                        pltpu.make_async_copy(vals_v[b2], s_sh.at[dst_v[b2]],
                                              scat_s[b2]).wait()
                        pltpu.make_async_copy(ones_v, d_sh.at[dst_v[b2]],
                                              scat_d[b2]).wait()

                    @pl.when(t + 2 < m)
                    def _(t=t, sb=sb, b2=b2):
                        pltpu.async_copy(src_slice(t + 2), src_v[sb],
                                         dma_s[sb])
                        pltpu.async_copy(dst_slice(t + 2), dst_v[b2],
                                         dma_d[b2])
                return _
            lax.fori_loop(0, n_groups, group, None)

            for b in range(4):
                for back in (1, 2):
                    @pl.when((m >= back) & ((m - back) % 4 == b))
                    def _(b=b):
                        pltpu.make_async_copy(vals_v[b], s_sh.at[dst_v[b]],
                                              scat_s[b]).wait()
                        pltpu.make_async_copy(ones_v, d_sh.at[dst_v[b]],
                                              scat_d[b]).wait()
            plsc.subcore_barrier()

            for acc, out in ((s_sh, s_out), (d_sh, d_out)):
                o = 0
                while o < zlen:
                    l = min(_CH, zlen - o)
                    pltpu.sync_copy(acc.at[pl.ds(zoff + o, l)],
                                    vals_v[0].at[pl.ds(0, l)])
                    pltpu.sync_copy(vals_v[0].at[pl.ds(0, l)],
                                    out.at[pl.ds(zoff + o, l)])
                    o += l

        @pl.when(cid == 0)
        def _():
            process(p0_hbm, e0_hbm, s0_out, d0_out)

        @pl.when(cid == 1)
        def _():
            process(p1_hbm, e1_hbm, s1_out, d1_out)

    f = pl.kernel(
        body,
        out_type=[jax.ShapeDtypeStruct((n_pad,), jnp.float32)] * 4,
        mesh=mesh,
        compiler_params=pltpu.CompilerParams(needs_layout_passes=False),
        scratch_types=[
            pltpu.VMEM((n,), jnp.float32),
            [pltpu.VMEM((_CH,), jnp.int32)] * 2,
            [pltpu.VMEM((_CH,), jnp.int32)] * 4,
            [pltpu.VMEM((_CH,), jnp.float32)] * 4,
            pltpu.VMEM((_CH,), jnp.float32),
            pltpu.VMEM_SHARED((n_pad,), jnp.float32),
            pltpu.VMEM_SHARED((n_pad,), jnp.float32),
            [pltpu.SemaphoreType.DMA] * 2,
            [pltpu.SemaphoreType.DMA] * 4,
            [pltpu.SemaphoreType.DMA] * 4,
            [pltpu.SemaphoreType.DMA] * 4,
        ],
    )
    return f(p0, p1, e0, e1)


def _tc_post(base, s0, d0, s1, d1, n):
    def body(base_ref, s0_ref, d0_ref, s1_ref, d1_ref, out_ref):
        one = jnp.float32(1.0)
        m0 = s0_ref[pl.ds(0, n)] / jnp.maximum(d0_ref[pl.ds(0, n)], one)
        m1 = s1_ref[pl.ds(0, n)] / jnp.maximum(d1_ref[pl.ds(0, n)], one)
        out_ref[...] = base_ref[...] + m0 + m1

    return pl.pallas_call(
        body,
        out_shape=jax.ShapeDtypeStruct((n,), jnp.float32),
    )(base, s0, d0, s1, d1)


def kernel(h, edge_index_rel0, edge_index_rel1,
           W_self0, W_neigh0, b0,
           W_self1, W_neigh1, b1,
           W_l, b_l):
    n, d = h.shape
    n_edges = edge_index_rel0.shape[1]
    n_pad = ((n + 127) // 128) * 128

    p0, p1, base = _tc_pre(h.T, W_self0, W_neigh0, b0, W_self1, W_neigh1, b1,
                           W_l, b_l, n)
    s0, d0, s1, d1 = _sc_edges(
        p0, p1, edge_index_rel0.reshape(-1), edge_index_rel1.reshape(-1),
        n, n_pad, n_edges)
    out = _tc_post(base, s0, d0, s1, d1, n)
    return out.reshape(n, 1)
